# baseline (device time: 15312 ns/iter reference)
import jax
import jax.numpy as jnp
from jax import lax
from jax.experimental import pallas as pl
from jax.experimental.pallas import tpu as pltpu

M = 512
N_OUT = 512
MQ = 256
SIZES = [32, 32, 32, 32, 32, 32, 32, 16, 8, 8]
OFFS = [sum(SIZES[:i]) for i in range(len(SIZES))]
C = len(SIZES)


def kernel(x):
    def body(x_ref, out_ref, recv_y_buf, recv_x_buf,
             y_send_sems, y_recv_sems, x_send_sems, x_recv_sems):
        my_x = lax.axis_index("x")
        my_y = lax.axis_index("y")
        other_x = 1 - my_x
        other_y = 1 - my_y

        barrier_sem = pltpu.get_barrier_semaphore()
        pl.semaphore_signal(barrier_sem, inc=1, device_id=(my_x, other_y),
                            device_id_type=pl.DeviceIdType.MESH)
        pl.semaphore_signal(barrier_sem, inc=1, device_id=(other_x, my_y),
                            device_id_type=pl.DeviceIdType.MESH)
        pl.semaphore_wait(barrier_sem, 2)

        y_rdmas = []
        for c in range(C):
            o, r = OFFS[c], SIZES[c]
            rdma = pltpu.make_async_remote_copy(
                src_ref=x_ref.at[0, pl.ds(my_x * MQ + o, r),
                                 pl.ds(other_y * N_OUT, N_OUT)],
                dst_ref=recv_y_buf.at[pl.ds(o, r)],
                send_sem=y_send_sems.at[c],
                recv_sem=y_recv_sems.at[c],
                device_id=(my_x, other_y),
                device_id_type=pl.DeviceIdType.MESH,
            )
            rdma.start()
            y_rdmas.append(rdma)

        x_rdmas = []
        for c in range(C):
            o, r = OFFS[c], SIZES[c]
            y_rdmas[c].wait_recv()
            rdma = pltpu.make_async_remote_copy(
                src_ref=recv_y_buf.at[pl.ds(o, r)],
                dst_ref=recv_x_buf.at[pl.ds(o, r)],
                send_sem=x_send_sems.at[c],
                recv_sem=x_recv_sems.at[c],
                device_id=(other_x, my_y),
                device_id_type=pl.DeviceIdType.MESH,
            )
            rdma.start()
            x_rdmas.append(rdma)
            out_ref[pl.ds(my_x * MQ + o, r), :] = (
                x_ref[0, pl.ds(my_x * MQ + o, r),
                      pl.ds(my_y * N_OUT, N_OUT)]
                + recv_y_buf[pl.ds(o, r), :]
            )

        for c in range(C):
            o, r = OFFS[c], SIZES[c]
            x_rdmas[c].wait_recv()
            out_ref[pl.ds(other_x * MQ + o, r), :] = (
                x_ref[0, pl.ds(other_x * MQ + o, r),
                      pl.ds(my_y * N_OUT, N_OUT)]
                + recv_x_buf[pl.ds(o, r), :]
            )

        for c in range(C):
            y_rdmas[c].wait_send()
            x_rdmas[c].wait_send()

    return pl.pallas_call(
        body,
        out_shape=jax.ShapeDtypeStruct((M, N_OUT), jnp.float32),
        in_specs=[pl.BlockSpec(memory_space=pltpu.VMEM)],
        out_specs=pl.BlockSpec(memory_space=pltpu.VMEM),
        scratch_shapes=[
            pltpu.VMEM((MQ, N_OUT), jnp.float32),
            pltpu.VMEM((MQ, N_OUT), jnp.float32),
            pltpu.SemaphoreType.DMA((C,)),
            pltpu.SemaphoreType.DMA((C,)),
            pltpu.SemaphoreType.DMA((C,)),
            pltpu.SemaphoreType.DMA((C,)),
        ],
        compiler_params=pltpu.CompilerParams(collective_id=0),
    )(x)


# device time: 15205 ns/iter; 1.0070x vs baseline; 1.0070x over previous
import jax
import jax.numpy as jnp
from jax import lax
from jax.experimental import pallas as pl
from jax.experimental.pallas import tpu as pltpu

M = 512
N_OUT = 512
MQ = 256
SIZES = [16] * 16
OFFS = [sum(SIZES[:i]) for i in range(len(SIZES))]
C = len(SIZES)


def kernel(x):
    def body(x_ref, out_ref, recv_y_buf, recv_x_buf,
             y_send_sems, y_recv_sems, x_send_sems, x_recv_sems):
        my_x = lax.axis_index("x")
        my_y = lax.axis_index("y")
        other_x = 1 - my_x
        other_y = 1 - my_y

        barrier_sem = pltpu.get_barrier_semaphore()
        pl.semaphore_signal(barrier_sem, inc=1, device_id=(my_x, other_y),
                            device_id_type=pl.DeviceIdType.MESH)
        pl.semaphore_signal(barrier_sem, inc=1, device_id=(other_x, my_y),
                            device_id_type=pl.DeviceIdType.MESH)
        pl.semaphore_wait(barrier_sem, 2)

        y_rdmas = []
        for c in range(C):
            o, r = OFFS[c], SIZES[c]
            rdma = pltpu.make_async_remote_copy(
                src_ref=x_ref.at[0, pl.ds(my_x * MQ + o, r),
                                 pl.ds(other_y * N_OUT, N_OUT)],
                dst_ref=recv_y_buf.at[pl.ds(o, r)],
                send_sem=y_send_sems.at[c],
                recv_sem=y_recv_sems.at[c],
                device_id=(my_x, other_y),
                device_id_type=pl.DeviceIdType.MESH,
            )
            rdma.start()
            y_rdmas.append(rdma)

        x_rdmas = []
        for c in range(C):
            o, r = OFFS[c], SIZES[c]
            y_rdmas[c].wait_recv()
            rdma = pltpu.make_async_remote_copy(
                src_ref=recv_y_buf.at[pl.ds(o, r)],
                dst_ref=recv_x_buf.at[pl.ds(o, r)],
                send_sem=x_send_sems.at[c],
                recv_sem=x_recv_sems.at[c],
                device_id=(other_x, my_y),
                device_id_type=pl.DeviceIdType.MESH,
            )
            rdma.start()
            x_rdmas.append(rdma)
            out_ref[pl.ds(my_x * MQ + o, r), :] = (
                x_ref[0, pl.ds(my_x * MQ + o, r),
                      pl.ds(my_y * N_OUT, N_OUT)]
                + recv_y_buf[pl.ds(o, r), :]
            )

        for c in range(C):
            o, r = OFFS[c], SIZES[c]
            x_rdmas[c].wait_recv()
            out_ref[pl.ds(other_x * MQ + o, r), :] = (
                x_ref[0, pl.ds(other_x * MQ + o, r),
                      pl.ds(my_y * N_OUT, N_OUT)]
                + recv_x_buf[pl.ds(o, r), :]
            )

        for c in range(C):
            y_rdmas[c].wait_send()
            x_rdmas[c].wait_send()

    return pl.pallas_call(
        body,
        out_shape=jax.ShapeDtypeStruct((M, N_OUT), jnp.float32),
        in_specs=[pl.BlockSpec(memory_space=pltpu.VMEM)],
        out_specs=pl.BlockSpec(memory_space=pltpu.VMEM),
        scratch_shapes=[
            pltpu.VMEM((MQ, N_OUT), jnp.float32),
            pltpu.VMEM((MQ, N_OUT), jnp.float32),
            pltpu.SemaphoreType.DMA((C,)),
            pltpu.SemaphoreType.DMA((C,)),
            pltpu.SemaphoreType.DMA((C,)),
            pltpu.SemaphoreType.DMA((C,)),
        ],
        compiler_params=pltpu.CompilerParams(collective_id=0),
    )(x)
